# Initial kernel scaffold; baseline (speedup 1.0000x reference)
import jax
import jax.numpy as jnp
from jax.experimental import pallas as pl


def kernel(xyz1, xyz2, features1, features2, W1, b1, g1, be1, W2, b2, g2, be2):
    raise NotImplementedError("write your pallas kernel here")


# speedup vs baseline: 13.0999x; 13.0999x over previous
import functools

import jax
import jax.numpy as jnp
from jax import lax
from jax.experimental import pallas as pl


def _p1(xyz1_r, xyz2t_r, f1_r, f2_r, w1a_r, w1b_r, b1_r, h1_r, st_r, *, S):
    x1 = xyz1_r[0]
    x2t = xyz2t_r[0]
    xa, xb, xc = x1[:, 0:1], x1[:, 1:2], x1[:, 2:3]
    ya, yb, yc = x2t[0:1, :], x2t[1:2, :], x2t[2:3, :]
    sq1 = xa * xa + xb * xb + xc * xc
    sq2 = ya * ya + yb * yb + yc * yc
    r16 = lambda v: v.astype(jnp.bfloat16).astype(jnp.float32)
    dot = r16(xa) * r16(ya) + r16(xb) * r16(yb) + r16(xc) * r16(yc)
    dist = jnp.maximum(sq1 + sq2 - 2.0 * dot, 0.0)

    bn = dist.shape[0]
    iota = lax.broadcasted_iota(jnp.int32, (bn, S), 1)
    a_mat = jnp.zeros((bn, S), jnp.float32)
    wsum = jnp.zeros((bn, 1), jnp.float32)
    for _ in range(3):
        m = jnp.min(dist, axis=1, keepdims=True)
        idx = jnp.min(jnp.where(dist == m, iota, S), axis=1, keepdims=True)
        sel = iota == idx
        w = 1.0 / (m + 1e-8)
        a_mat = a_mat + jnp.where(sel, w, 0.0)
        wsum = wsum + w
        dist = jnp.where(sel, jnp.inf, dist)
    a_mat = a_mat / wsum

    interp = jnp.dot(a_mat, f2_r[0], preferred_element_type=jnp.float32, precision=lax.Precision.HIGHEST)
    h1 = (jnp.dot(interp, w1a_r[...], preferred_element_type=jnp.float32, precision=lax.Precision.HIGHEST)
          + jnp.dot(f1_r[0], w1b_r[...], preferred_element_type=jnp.float32, precision=lax.Precision.HIGHEST)
          + b1_r[...])
    h1_r[0] = h1

    first = jnp.logical_and(pl.program_id(0) == 0, pl.program_id(1) == 0)

    @pl.when(first)
    def _():
        st_r[...] = jnp.zeros_like(st_r)

    st_r[0:1, :] = st_r[0:1, :] + jnp.sum(h1, axis=0, keepdims=True)
    st_r[1:2, :] = st_r[1:2, :] + jnp.sum(h1 * h1, axis=0, keepdims=True)


def _p2(h1_r, st_r, g_r, be_r, w2_r, b2_r, h2_r, st2_r, *, count):
    inv = 1.0 / count
    mean = st_r[0:1, :] * inv
    var = st_r[1:2, :] * inv - mean * mean
    scale = g_r[...] * lax.rsqrt(var + 1e-5)
    shift = be_r[...] - mean * scale
    a1 = jnp.maximum(h1_r[...] * scale + shift, 0.0)
    h2 = jnp.dot(a1, w2_r[...], preferred_element_type=jnp.float32, precision=lax.Precision.HIGHEST) + b2_r[...]
    h2_r[...] = h2

    @pl.when(pl.program_id(0) == 0)
    def _():
        st2_r[...] = jnp.zeros_like(st2_r)

    st2_r[0:1, :] = st2_r[0:1, :] + jnp.sum(h2, axis=0, keepdims=True)
    st2_r[1:2, :] = st2_r[1:2, :] + jnp.sum(h2 * h2, axis=0, keepdims=True)


def _p3(h2_r, st_r, g_r, be_r, o_r, *, count):
    inv = 1.0 / count
    mean = st_r[0:1, :] * inv
    var = st_r[1:2, :] * inv - mean * mean
    scale = g_r[...] * lax.rsqrt(var + 1e-5)
    shift = be_r[...] - mean * scale
    o_r[...] = jnp.maximum(h2_r[...] * scale + shift, 0.0)


def kernel(xyz1, xyz2, features1, features2, W1, b1, g1, be1, W2, b2, g2, be2):
    B, N, _ = xyz1.shape
    S = xyz2.shape[1]
    C1 = features1.shape[2]
    C2 = features2.shape[2]
    H = W1.shape[0]
    f32 = jnp.float32

    xyz2t = jnp.swapaxes(xyz2, 1, 2)
    w1a = W1[:, :C2].T
    w1b = W1[:, C2:].T
    w2t = W2.T

    BN_BLK = 512
    NB = N // BN_BLK

    h1, st1 = pl.pallas_call(
        functools.partial(_p1, S=S),
        grid=(B, NB),
        in_specs=[
            pl.BlockSpec((1, BN_BLK, 3), lambda b, i: (b, i, 0)),
            pl.BlockSpec((1, 3, S), lambda b, i: (b, 0, 0)),
            pl.BlockSpec((1, BN_BLK, C1), lambda b, i: (b, i, 0)),
            pl.BlockSpec((1, S, C2), lambda b, i: (b, 0, 0)),
            pl.BlockSpec((C2, H), lambda b, i: (0, 0)),
            pl.BlockSpec((C1, H), lambda b, i: (0, 0)),
            pl.BlockSpec((1, H), lambda b, i: (0, 0)),
        ],
        out_specs=[
            pl.BlockSpec((1, BN_BLK, H), lambda b, i: (b, i, 0)),
            pl.BlockSpec((2, H), lambda b, i: (0, 0)),
        ],
        out_shape=[
            jax.ShapeDtypeStruct((B, N, H), f32),
            jax.ShapeDtypeStruct((2, H), f32),
        ],
    )(xyz1, xyz2t, features1, features2, w1a, w1b, b1.reshape(1, H))

    R = B * N
    RB = 2048
    h1f = h1.reshape(R, H)
    count = float(R)

    h2, st2 = pl.pallas_call(
        functools.partial(_p2, count=count),
        grid=(R // RB,),
        in_specs=[
            pl.BlockSpec((RB, H), lambda i: (i, 0)),
            pl.BlockSpec((2, H), lambda i: (0, 0)),
            pl.BlockSpec((1, H), lambda i: (0, 0)),
            pl.BlockSpec((1, H), lambda i: (0, 0)),
            pl.BlockSpec((H, H), lambda i: (0, 0)),
            pl.BlockSpec((1, H), lambda i: (0, 0)),
        ],
        out_specs=[
            pl.BlockSpec((RB, H), lambda i: (i, 0)),
            pl.BlockSpec((2, H), lambda i: (0, 0)),
        ],
        out_shape=[
            jax.ShapeDtypeStruct((R, H), f32),
            jax.ShapeDtypeStruct((2, H), f32),
        ],
    )(h1f, st1, g1.reshape(1, H), be1.reshape(1, H), w2t, b2.reshape(1, H))

    out = pl.pallas_call(
        functools.partial(_p3, count=count),
        grid=(R // RB,),
        in_specs=[
            pl.BlockSpec((RB, H), lambda i: (i, 0)),
            pl.BlockSpec((2, H), lambda i: (0, 0)),
            pl.BlockSpec((1, H), lambda i: (0, 0)),
            pl.BlockSpec((1, H), lambda i: (0, 0)),
        ],
        out_specs=pl.BlockSpec((RB, H), lambda i: (i, 0)),
        out_shape=jax.ShapeDtypeStruct((R, H), f32),
    )(h2, st2, g2.reshape(1, H), be2.reshape(1, H))

    return out.reshape(B, N, H)


# speedup vs baseline: 13.5048x; 1.0309x over previous
import functools

import jax
import jax.numpy as jnp
from jax import lax
from jax.experimental import pallas as pl
from jax.experimental.pallas import tpu as pltpu, tpu_sc as plsc


def _p1(xyz1_r, xyz2t_r, f1_r, w1b_r, b1_r, idx_r, w_r, h1p_r, *, S):
    x1 = xyz1_r[0]
    x2t = xyz2t_r[0]
    xa, xb, xc = x1[:, 0:1], x1[:, 1:2], x1[:, 2:3]
    ya, yb, yc = x2t[0:1, :], x2t[1:2, :], x2t[2:3, :]
    sq1 = xa * xa + xb * xb + xc * xc
    sq2 = ya * ya + yb * yb + yc * yc
    dot2 = jnp.dot(x1 * -2.0, x2t, preferred_element_type=jnp.float32)
    dist = jnp.maximum(sq1 + sq2 + dot2, 0.0)

    bn = dist.shape[0]
    iota = lax.broadcasted_iota(jnp.int32, (bn, S), 1).astype(jnp.float32)
    big = jnp.float32(float(S))
    idxs = []
    ws = []
    for k in range(3):
        m = jnp.min(dist, axis=1, keepdims=True)
        idx = jnp.min(jnp.where(dist == m, iota, big), axis=1, keepdims=True)
        idxs.append(idx)
        ws.append(1.0 / (m + 1e-8))
        if k < 2:
            dist = jnp.where(iota == idx, jnp.inf, dist)
    wsum = ws[0] + ws[1] + ws[2]
    b = pl.program_id(0)
    for k in range(3):
        idx_r[0, :, k:k + 1] = idxs[k].astype(jnp.int32) + b * S
        w_r[0, :, k:k + 1] = ws[k] / wsum

    h1p = jnp.dot(f1_r[0], w1b_r[...], preferred_element_type=jnp.float32) + b1_r[...]
    h1p_r[0] = h1p


def _sc_gather(f2_hbm, idx_hbm, w16_hbm, out_hbm, idx_v, w_v, rows_v, acc_v, sem, *,
               pts_per_blk, n_blk, C2, NC):
    wid = lax.axis_index("s") * NC + lax.axis_index("c")

    def body(blk, carry):
        base = (wid * n_blk + blk) * pts_per_blk
        g = pts_per_blk * 3
        pltpu.sync_copy(idx_hbm.at[pl.ds(base * 3, g)], idx_v)
        pltpu.sync_copy(w16_hbm.at[pl.ds(base * 3, g)], w_v)
        pltpu.async_copy(f2_hbm.at[idx_v], rows_v, sem).wait()
        for p in range(pts_per_blk):
            w0 = w_v[3 * p]
            w1 = w_v[3 * p + 1]
            w2 = w_v[3 * p + 2]
            for c in range(C2 // 16):
                sl = pl.ds(c * 16, 16)
                acc_v[p, sl] = (rows_v[3 * p, sl] * w0
                                + rows_v[3 * p + 1, sl] * w1
                                + rows_v[3 * p + 2, sl] * w2)
        pltpu.sync_copy(acc_v, out_hbm.at[pl.ds(base, pts_per_blk)])
        return carry

    lax.fori_loop(0, n_blk, body, 0)


def _p1b(interp_r, h1p_r, w1a_r, h1_r, st_r):
    h1 = jnp.dot(interp_r[...], w1a_r[...], preferred_element_type=jnp.float32) \
        + h1p_r[...]
    h1_r[...] = h1.astype(jnp.bfloat16)

    @pl.when(pl.program_id(0) == 0)
    def _():
        st_r[...] = jnp.zeros_like(st_r)

    st_r[0:1, :] = st_r[0:1, :] + jnp.sum(h1, axis=0, keepdims=True)
    st_r[1:2, :] = st_r[1:2, :] + jnp.sum(h1 * h1, axis=0, keepdims=True)


def _p2(h1_r, st_r, g_r, be_r, w2_r, b2_r, h2_r, st2_r, *, count):
    inv = 1.0 / count
    mean = st_r[0:1, :] * inv
    var = st_r[1:2, :] * inv - mean * mean
    scale = g_r[...] * lax.rsqrt(var + 1e-5)
    shift = be_r[...] - mean * scale
    a1 = jnp.maximum(h1_r[...].astype(jnp.float32) * scale + shift, 0.0)
    h2 = jnp.dot(a1, w2_r[...], preferred_element_type=jnp.float32) + b2_r[...]
    h2_r[...] = h2.astype(jnp.bfloat16)

    @pl.when(pl.program_id(0) == 0)
    def _():
        st2_r[...] = jnp.zeros_like(st2_r)

    st2_r[0:1, :] = st2_r[0:1, :] + jnp.sum(h2, axis=0, keepdims=True)
    st2_r[1:2, :] = st2_r[1:2, :] + jnp.sum(h2 * h2, axis=0, keepdims=True)


def _p3(h2_r, st_r, g_r, be_r, o_r, *, count):
    inv = 1.0 / count
    mean = st_r[0:1, :] * inv
    var = st_r[1:2, :] * inv - mean * mean
    scale = g_r[...] * lax.rsqrt(var + 1e-5)
    shift = be_r[...] - mean * scale
    o_r[...] = jnp.maximum(h2_r[...].astype(jnp.float32) * scale + shift, 0.0)


def kernel(xyz1, xyz2, features1, features2, W1, b1, g1, be1, W2, b2, g2, be2):
    B, N, _ = xyz1.shape
    S = xyz2.shape[1]
    C1 = features1.shape[2]
    C2 = features2.shape[2]
    H = W1.shape[0]
    f32 = jnp.float32

    xyz2t = jnp.swapaxes(xyz2, 1, 2)
    w1a = W1[:, :C2].T
    w1b = W1[:, C2:].T
    w2t = W2.T

    BN_BLK = 512
    NB = N // BN_BLK

    idx, w, h1p = pl.pallas_call(
        functools.partial(_p1, S=S),
        grid=(B, NB),
        in_specs=[
            pl.BlockSpec((1, BN_BLK, 3), lambda b, i: (b, i, 0)),
            pl.BlockSpec((1, 3, S), lambda b, i: (b, 0, 0)),
            pl.BlockSpec((1, BN_BLK, C1), lambda b, i: (b, i, 0)),
            pl.BlockSpec((C1, H), lambda b, i: (0, 0)),
            pl.BlockSpec((1, H), lambda b, i: (0, 0)),
        ],
        out_specs=[
            pl.BlockSpec((1, BN_BLK, 4), lambda b, i: (b, i, 0)),
            pl.BlockSpec((1, BN_BLK, 4), lambda b, i: (b, i, 0)),
            pl.BlockSpec((1, BN_BLK, H), lambda b, i: (b, i, 0)),
        ],
        out_shape=[
            jax.ShapeDtypeStruct((B, N, 4), jnp.int32),
            jax.ShapeDtypeStruct((B, N, 4), f32),
            jax.ShapeDtypeStruct((B, N, H), f32),
        ],
    )(xyz1, xyz2t, features1, w1b, b1.reshape(1, H))

    R = B * N
    info = plsc.get_sparse_core_info()
    NC, NS = info.num_cores, info.num_subcores
    NW = NC * NS
    PTS_BLK = 32
    n_blk = R // (NW * PTS_BLK)

    idx3 = idx[..., :3].reshape(R * 3)
    w16 = jnp.broadcast_to(w[..., :3].reshape(R * 3)[:, None], (R * 3, 16))
    f2flat = features2.reshape(B * S, C2)

    mesh = plsc.VectorSubcoreMesh(core_axis_name="c", subcore_axis_name="s")
    interp = pl.kernel(
        functools.partial(_sc_gather, pts_per_blk=PTS_BLK, n_blk=n_blk,
                          C2=C2, NC=NC),
        mesh=mesh,
        out_type=jax.ShapeDtypeStruct((R, C2), f32),
        scratch_types=[
            pltpu.VMEM((PTS_BLK * 3,), jnp.int32),
            pltpu.VMEM((PTS_BLK * 3, 16), f32),
            pltpu.VMEM((PTS_BLK * 3, C2), f32),
            pltpu.VMEM((PTS_BLK, C2), f32),
            pltpu.SemaphoreType.DMA,
        ],
    )(f2flat, idx3, w16)

    RB = 2048
    count = float(R)

    h1, st1 = pl.pallas_call(
        _p1b,
        grid=(R // RB,),
        in_specs=[
            pl.BlockSpec((RB, C2), lambda i: (i, 0)),
            pl.BlockSpec((RB, H), lambda i: (i, 0)),
            pl.BlockSpec((C2, H), lambda i: (0, 0)),
        ],
        out_specs=[
            pl.BlockSpec((RB, H), lambda i: (i, 0)),
            pl.BlockSpec((2, H), lambda i: (0, 0)),
        ],
        out_shape=[
            jax.ShapeDtypeStruct((R, H), jnp.bfloat16),
            jax.ShapeDtypeStruct((2, H), f32),
        ],
    )(interp, h1p.reshape(R, H), w1a)

    h2, st2 = pl.pallas_call(
        functools.partial(_p2, count=count),
        grid=(R // RB,),
        in_specs=[
            pl.BlockSpec((RB, H), lambda i: (i, 0)),
            pl.BlockSpec((2, H), lambda i: (0, 0)),
            pl.BlockSpec((1, H), lambda i: (0, 0)),
            pl.BlockSpec((1, H), lambda i: (0, 0)),
            pl.BlockSpec((H, H), lambda i: (0, 0)),
            pl.BlockSpec((1, H), lambda i: (0, 0)),
        ],
        out_specs=[
            pl.BlockSpec((RB, H), lambda i: (i, 0)),
            pl.BlockSpec((2, H), lambda i: (0, 0)),
        ],
        out_shape=[
            jax.ShapeDtypeStruct((R, H), jnp.bfloat16),
            jax.ShapeDtypeStruct((2, H), f32),
        ],
    )(h1, st1, g1.reshape(1, H), be1.reshape(1, H), w2t, b2.reshape(1, H))

    out = pl.pallas_call(
        functools.partial(_p3, count=count),
        grid=(R // RB,),
        in_specs=[
            pl.BlockSpec((RB, H), lambda i: (i, 0)),
            pl.BlockSpec((2, H), lambda i: (0, 0)),
            pl.BlockSpec((1, H), lambda i: (0, 0)),
            pl.BlockSpec((1, H), lambda i: (0, 0)),
        ],
        out_specs=pl.BlockSpec((RB, H), lambda i: (i, 0)),
        out_shape=jax.ShapeDtypeStruct((R, H), f32),
    )(h2, st2, g2.reshape(1, H), be2.reshape(1, H))

    return out.reshape(B, N, H)


# speedup vs baseline: 35.0675x; 2.5967x over previous
import functools

import jax
import jax.numpy as jnp
from jax import lax
from jax.experimental import pallas as pl


def _p1(xyz1_r, xyz2t_r, f1_r, f2_r, w1a_r, w1b_r, b1_r, h1_r, st_r, *, S):
    x1 = xyz1_r[0]
    x2t = xyz2t_r[0]
    xa, xb, xc = x1[:, 0:1], x1[:, 1:2], x1[:, 2:3]
    ya, yb, yc = x2t[0:1, :], x2t[1:2, :], x2t[2:3, :]
    sq1 = xa * xa + xb * xb + xc * xc
    sq2 = ya * ya + yb * yb + yc * yc
    dot2 = jnp.dot(x1 * -2.0, x2t, preferred_element_type=jnp.float32)
    dist = jnp.maximum(sq1 + sq2 + dot2, 0.0)

    bn = dist.shape[0]
    inf = jnp.float32(jnp.inf)

    LW = 128
    nch = S // LW
    a = dist[:, 0:LW]
    b = jnp.full_like(a, inf)
    c = jnp.full_like(a, inf)
    for j in range(1, nch):
        d = dist[:, j * LW:(j + 1) * LW]
        lo = jnp.minimum(a, d)
        hi = jnp.maximum(a, d)
        a = lo
        lo2 = jnp.minimum(b, hi)
        hi2 = jnp.maximum(b, hi)
        b = lo2
        c = jnp.minimum(c, hi2)

    i128 = lax.broadcasted_iota(jnp.int32, (bn, LW), 1).astype(jnp.float32)
    big = jnp.float32(float(S))
    m1 = jnp.min(a, axis=1, keepdims=True)
    p1 = jnp.min(jnp.where(a == m1, i128, big), axis=1, keepdims=True)
    rm1 = i128 == p1
    a1 = jnp.where(rm1, b, a)
    b1v = jnp.where(rm1, c, b)
    m2 = jnp.min(a1, axis=1, keepdims=True)
    p2 = jnp.min(jnp.where(a1 == m2, i128, big), axis=1, keepdims=True)
    rm2 = i128 == p2
    a2 = jnp.where(rm2, b1v, a1)
    m3 = jnp.min(a2, axis=1, keepdims=True)

    w1 = 1.0 / (m1 + 1e-8)
    w2 = 1.0 / (m2 + 1e-8)
    w3 = 1.0 / (m3 + 1e-8)
    wsum = w1 + w2 + w3
    nw1, nw2, nw3 = w1 / wsum, w2 / wsum, w3 / wsum

    a_mat = jnp.where(dist == m1, nw1, jnp.where(
        dist == m2, nw2, jnp.where(dist == m3, nw3, 0.0)))

    interp = jnp.dot(a_mat, f2_r[0], preferred_element_type=jnp.float32)
    h1 = (jnp.dot(interp, w1a_r[...], preferred_element_type=jnp.float32)
          + jnp.dot(f1_r[0], w1b_r[...], preferred_element_type=jnp.float32)
          + b1_r[...])
    h1_r[0] = h1.astype(jnp.bfloat16)

    first = jnp.logical_and(pl.program_id(0) == 0, pl.program_id(1) == 0)

    @pl.when(first)
    def _():
        st_r[...] = jnp.zeros_like(st_r)

    st_r[0:1, :] = st_r[0:1, :] + jnp.sum(h1, axis=0, keepdims=True)
    st_r[1:2, :] = st_r[1:2, :] + jnp.sum(h1 * h1, axis=0, keepdims=True)


def _p2(h1_r, st_r, g_r, be_r, w2_r, b2_r, h2_r, st2_r, *, count):
    inv = 1.0 / count
    mean = st_r[0:1, :] * inv
    var = st_r[1:2, :] * inv - mean * mean
    scale = g_r[...] * lax.rsqrt(var + 1e-5)
    shift = be_r[...] - mean * scale
    a1 = jnp.maximum(h1_r[...].astype(jnp.float32) * scale + shift, 0.0)
    h2 = jnp.dot(a1, w2_r[...], preferred_element_type=jnp.float32) + b2_r[...]
    h2_r[...] = h2.astype(jnp.bfloat16)

    @pl.when(pl.program_id(0) == 0)
    def _():
        st2_r[...] = jnp.zeros_like(st2_r)

    st2_r[0:1, :] = st2_r[0:1, :] + jnp.sum(h2, axis=0, keepdims=True)
    st2_r[1:2, :] = st2_r[1:2, :] + jnp.sum(h2 * h2, axis=0, keepdims=True)


def _p3(h2_r, st_r, g_r, be_r, o_r, *, count):
    inv = 1.0 / count
    mean = st_r[0:1, :] * inv
    var = st_r[1:2, :] * inv - mean * mean
    scale = g_r[...] * lax.rsqrt(var + 1e-5)
    shift = be_r[...] - mean * scale
    o_r[...] = jnp.maximum(h2_r[...].astype(jnp.float32) * scale + shift, 0.0)


def kernel(xyz1, xyz2, features1, features2, W1, b1, g1, be1, W2, b2, g2, be2):
    B, N, _ = xyz1.shape
    S = xyz2.shape[1]
    C1 = features1.shape[2]
    C2 = features2.shape[2]
    H = W1.shape[0]
    f32 = jnp.float32

    xyz2t = jnp.swapaxes(xyz2, 1, 2)
    w1a = W1[:, :C2].T
    w1b = W1[:, C2:].T
    w2t = W2.T

    BN_BLK = 512
    NB = N // BN_BLK

    h1, st1 = pl.pallas_call(
        functools.partial(_p1, S=S),
        grid=(B, NB),
        in_specs=[
            pl.BlockSpec((1, BN_BLK, 3), lambda b, i: (b, i, 0)),
            pl.BlockSpec((1, 3, S), lambda b, i: (b, 0, 0)),
            pl.BlockSpec((1, BN_BLK, C1), lambda b, i: (b, i, 0)),
            pl.BlockSpec((1, S, C2), lambda b, i: (b, 0, 0)),
            pl.BlockSpec((C2, H), lambda b, i: (0, 0)),
            pl.BlockSpec((C1, H), lambda b, i: (0, 0)),
            pl.BlockSpec((1, H), lambda b, i: (0, 0)),
        ],
        out_specs=[
            pl.BlockSpec((1, BN_BLK, H), lambda b, i: (b, i, 0)),
            pl.BlockSpec((2, H), lambda b, i: (0, 0)),
        ],
        out_shape=[
            jax.ShapeDtypeStruct((B, N, H), jnp.bfloat16),
            jax.ShapeDtypeStruct((2, H), f32),
        ],
    )(xyz1, xyz2t, features1, features2, w1a, w1b, b1.reshape(1, H))

    R = B * N
    RB = 2048
    h1f = h1.reshape(R, H)
    count = float(R)

    h2, st2 = pl.pallas_call(
        functools.partial(_p2, count=count),
        grid=(R // RB,),
        in_specs=[
            pl.BlockSpec((RB, H), lambda i: (i, 0)),
            pl.BlockSpec((2, H), lambda i: (0, 0)),
            pl.BlockSpec((1, H), lambda i: (0, 0)),
            pl.BlockSpec((1, H), lambda i: (0, 0)),
            pl.BlockSpec((H, H), lambda i: (0, 0)),
            pl.BlockSpec((1, H), lambda i: (0, 0)),
        ],
        out_specs=[
            pl.BlockSpec((RB, H), lambda i: (i, 0)),
            pl.BlockSpec((2, H), lambda i: (0, 0)),
        ],
        out_shape=[
            jax.ShapeDtypeStruct((R, H), jnp.bfloat16),
            jax.ShapeDtypeStruct((2, H), f32),
        ],
    )(h1f, st1, g1.reshape(1, H), be1.reshape(1, H), w2t, b2.reshape(1, H))

    out = pl.pallas_call(
        functools.partial(_p3, count=count),
        grid=(R // RB,),
        in_specs=[
            pl.BlockSpec((RB, H), lambda i: (i, 0)),
            pl.BlockSpec((2, H), lambda i: (0, 0)),
            pl.BlockSpec((1, H), lambda i: (0, 0)),
            pl.BlockSpec((1, H), lambda i: (0, 0)),
        ],
        out_specs=pl.BlockSpec((RB, H), lambda i: (i, 0)),
        out_shape=jax.ShapeDtypeStruct((R, H), f32),
    )(h2, st2, g2.reshape(1, H), be2.reshape(1, H))

    return out.reshape(B, N, H)


# speedup vs baseline: 36.7949x; 1.0493x over previous
import functools

import jax
import jax.numpy as jnp
from jax import lax
from jax.experimental import pallas as pl


def _p1(xyz1_r, xyz2t_r, f1_r, f2_r, w1a_r, w1b_r, b1_r, h1_r, st_r, *, S):
    x1 = xyz1_r[0]
    x2t = xyz2t_r[0]
    xa, xb, xc = x1[:, 0:1], x1[:, 1:2], x1[:, 2:3]
    ya, yb, yc = x2t[0:1, :], x2t[1:2, :], x2t[2:3, :]
    sq1 = xa * xa + xb * xb + xc * xc
    sq2 = ya * ya + yb * yb + yc * yc
    dot2 = jnp.dot(x1 * -2.0, x2t, preferred_element_type=jnp.float32)
    dist = jnp.maximum(sq1 + sq2 + dot2, 0.0)

    bn = dist.shape[0]
    inf = jnp.float32(jnp.inf)

    LW = 128
    nch = S // LW
    a = dist[:, 0:LW]
    b = jnp.full_like(a, inf)
    c = jnp.full_like(a, inf)
    for j in range(1, nch):
        d = dist[:, j * LW:(j + 1) * LW]
        lo = jnp.minimum(a, d)
        hi = jnp.maximum(a, d)
        a = lo
        lo2 = jnp.minimum(b, hi)
        hi2 = jnp.maximum(b, hi)
        b = lo2
        c = jnp.minimum(c, hi2)

    i128 = lax.broadcasted_iota(jnp.int32, (bn, LW), 1).astype(jnp.float32)
    big = jnp.float32(float(S))
    m1 = jnp.min(a, axis=1, keepdims=True)
    p1 = jnp.min(jnp.where(a == m1, i128, big), axis=1, keepdims=True)
    rm1 = i128 == p1
    a1 = jnp.where(rm1, b, a)
    b1v = jnp.where(rm1, c, b)
    m2 = jnp.min(a1, axis=1, keepdims=True)
    p2 = jnp.min(jnp.where(a1 == m2, i128, big), axis=1, keepdims=True)
    rm2 = i128 == p2
    a2 = jnp.where(rm2, b1v, a1)
    m3 = jnp.min(a2, axis=1, keepdims=True)

    w1 = 1.0 / (m1 + 1e-8)
    w2 = 1.0 / (m2 + 1e-8)
    w3 = 1.0 / (m3 + 1e-8)
    wsum = w1 + w2 + w3
    nw1, nw2, nw3 = w1 / wsum, w2 / wsum, w3 / wsum

    a_mat = jnp.where(dist == m1, nw1, jnp.where(
        dist == m2, nw2, jnp.where(dist == m3, nw3, 0.0)))

    interp = jnp.dot(a_mat, f2_r[0], preferred_element_type=jnp.float32)
    h1 = (jnp.dot(interp, w1a_r[...], preferred_element_type=jnp.float32)
          + jnp.dot(f1_r[0], w1b_r[...], preferred_element_type=jnp.float32)
          + b1_r[...])
    h1_r[0] = h1.astype(jnp.bfloat16)

    first = jnp.logical_and(pl.program_id(0) == 0, pl.program_id(1) == 0)

    @pl.when(first)
    def _():
        st_r[...] = jnp.zeros_like(st_r)

    st_r[0:1, :] = st_r[0:1, :] + jnp.sum(h1, axis=0, keepdims=True)
    st_r[1:2, :] = st_r[1:2, :] + jnp.sum(h1 * h1, axis=0, keepdims=True)


def _p2(h1_r, st_r, g_r, be_r, w2_r, b2_r, h2_r, st2_r, *, count):
    inv = 1.0 / count
    mean = st_r[0:1, :] * inv
    var = st_r[1:2, :] * inv - mean * mean
    scale = g_r[...] * lax.rsqrt(var + 1e-5)
    shift = be_r[...] - mean * scale
    a1 = jnp.maximum(h1_r[...].astype(jnp.float32) * scale + shift, 0.0)
    h2 = jnp.dot(a1, w2_r[...], preferred_element_type=jnp.float32) + b2_r[...]
    h2_r[...] = h2.astype(jnp.bfloat16)

    @pl.when(pl.program_id(0) == 0)
    def _():
        st2_r[...] = jnp.zeros_like(st2_r)

    st2_r[0:1, :] = st2_r[0:1, :] + jnp.sum(h2, axis=0, keepdims=True)
    st2_r[1:2, :] = st2_r[1:2, :] + jnp.sum(h2 * h2, axis=0, keepdims=True)


def _p3(h2_r, st_r, g_r, be_r, o_r, *, count):
    inv = 1.0 / count
    mean = st_r[0:1, :] * inv
    var = st_r[1:2, :] * inv - mean * mean
    scale = g_r[...] * lax.rsqrt(var + 1e-5)
    shift = be_r[...] - mean * scale
    o_r[...] = jnp.maximum(h2_r[...].astype(jnp.float32) * scale + shift, 0.0)


def kernel(xyz1, xyz2, features1, features2, W1, b1, g1, be1, W2, b2, g2, be2):
    B, N, _ = xyz1.shape
    S = xyz2.shape[1]
    C1 = features1.shape[2]
    C2 = features2.shape[2]
    H = W1.shape[0]
    f32 = jnp.float32

    xyz2t = jnp.swapaxes(xyz2, 1, 2)
    w1a = W1[:, :C2].T
    w1b = W1[:, C2:].T
    w2t = W2.T

    BN_BLK = 512
    NB = N // BN_BLK

    h1, st1 = pl.pallas_call(
        functools.partial(_p1, S=S),
        grid=(B, NB),
        in_specs=[
            pl.BlockSpec((1, BN_BLK, 3), lambda b, i: (b, i, 0)),
            pl.BlockSpec((1, 3, S), lambda b, i: (b, 0, 0)),
            pl.BlockSpec((1, BN_BLK, C1), lambda b, i: (b, i, 0)),
            pl.BlockSpec((1, S, C2), lambda b, i: (b, 0, 0)),
            pl.BlockSpec((C2, H), lambda b, i: (0, 0)),
            pl.BlockSpec((C1, H), lambda b, i: (0, 0)),
            pl.BlockSpec((1, H), lambda b, i: (0, 0)),
        ],
        out_specs=[
            pl.BlockSpec((1, BN_BLK, H), lambda b, i: (b, i, 0)),
            pl.BlockSpec((2, H), lambda b, i: (0, 0)),
        ],
        out_shape=[
            jax.ShapeDtypeStruct((B, N, H), jnp.bfloat16),
            jax.ShapeDtypeStruct((2, H), f32),
        ],
    )(xyz1, xyz2t, features1, features2, w1a, w1b, b1.reshape(1, H))

    R = B * N
    RB = 4096
    h1f = h1.reshape(R, H)
    count = float(R)

    h2, st2 = pl.pallas_call(
        functools.partial(_p2, count=count),
        grid=(R // RB,),
        in_specs=[
            pl.BlockSpec((RB, H), lambda i: (i, 0)),
            pl.BlockSpec((2, H), lambda i: (0, 0)),
            pl.BlockSpec((1, H), lambda i: (0, 0)),
            pl.BlockSpec((1, H), lambda i: (0, 0)),
            pl.BlockSpec((H, H), lambda i: (0, 0)),
            pl.BlockSpec((1, H), lambda i: (0, 0)),
        ],
        out_specs=[
            pl.BlockSpec((RB, H), lambda i: (i, 0)),
            pl.BlockSpec((2, H), lambda i: (0, 0)),
        ],
        out_shape=[
            jax.ShapeDtypeStruct((R, H), jnp.bfloat16),
            jax.ShapeDtypeStruct((2, H), f32),
        ],
    )(h1f, st1, g1.reshape(1, H), be1.reshape(1, H), w2t, b2.reshape(1, H))

    out = pl.pallas_call(
        functools.partial(_p3, count=count),
        grid=(R // RB,),
        in_specs=[
            pl.BlockSpec((RB, H), lambda i: (i, 0)),
            pl.BlockSpec((2, H), lambda i: (0, 0)),
            pl.BlockSpec((1, H), lambda i: (0, 0)),
            pl.BlockSpec((1, H), lambda i: (0, 0)),
        ],
        out_specs=pl.BlockSpec((RB, H), lambda i: (i, 0)),
        out_shape=jax.ShapeDtypeStruct((R, H), f32),
    )(h2, st2, g2.reshape(1, H), be2.reshape(1, H))

    return out.reshape(B, N, H)


# speedup vs baseline: 37.1411x; 1.0094x over previous
import functools

import jax
import jax.numpy as jnp
from jax import lax
from jax.experimental import pallas as pl


def _p1(xyz1_r, xyz2t_r, f1_r, f2_r, w1a_r, w1b_r, b1_r, h1_r, st_r, *, S):
    x1 = xyz1_r[0]
    x2t = xyz2t_r[0]
    xa, xb, xc = x1[:, 0:1], x1[:, 1:2], x1[:, 2:3]
    ya, yb, yc = x2t[0:1, :], x2t[1:2, :], x2t[2:3, :]
    sq1 = xa * xa + xb * xb + xc * xc
    sq2 = ya * ya + yb * yb + yc * yc
    dot2 = jnp.dot(x1 * -2.0, x2t, preferred_element_type=jnp.float32)
    dist = jnp.maximum(sq1 + sq2 + dot2, 0.0)

    bn = dist.shape[0]
    inf = jnp.float32(jnp.inf)

    LW = 128
    nch = S // LW
    a = dist[:, 0:LW]
    b = jnp.full_like(a, inf)
    c = jnp.full_like(a, inf)
    for j in range(1, nch):
        d = dist[:, j * LW:(j + 1) * LW]
        lo = jnp.minimum(a, d)
        hi = jnp.maximum(a, d)
        a = lo
        lo2 = jnp.minimum(b, hi)
        hi2 = jnp.maximum(b, hi)
        b = lo2
        c = jnp.minimum(c, hi2)

    i128 = lax.broadcasted_iota(jnp.int32, (bn, LW), 1).astype(jnp.float32)
    big = jnp.float32(float(S))
    m1 = jnp.min(a, axis=1, keepdims=True)
    p1 = jnp.min(jnp.where(a == m1, i128, big), axis=1, keepdims=True)
    rm1 = i128 == p1
    a1 = jnp.where(rm1, b, a)
    b1v = jnp.where(rm1, c, b)
    m2 = jnp.min(a1, axis=1, keepdims=True)
    p2 = jnp.min(jnp.where(a1 == m2, i128, big), axis=1, keepdims=True)
    rm2 = i128 == p2
    a2 = jnp.where(rm2, b1v, a1)
    m3 = jnp.min(a2, axis=1, keepdims=True)

    w1 = 1.0 / (m1 + 1e-8)
    w2 = 1.0 / (m2 + 1e-8)
    w3 = 1.0 / (m3 + 1e-8)
    wsum = w1 + w2 + w3
    nw1, nw2, nw3 = w1 / wsum, w2 / wsum, w3 / wsum

    a_mat = jnp.where(dist == m1, nw1, jnp.where(
        dist == m2, nw2, jnp.where(dist == m3, nw3, 0.0)))

    interp = jnp.dot(a_mat, f2_r[0], preferred_element_type=jnp.float32)
    h1 = (jnp.dot(interp, w1a_r[...], preferred_element_type=jnp.float32)
          + jnp.dot(f1_r[0], w1b_r[...], preferred_element_type=jnp.float32)
          + b1_r[...])
    h1_r[0] = h1.astype(jnp.bfloat16)

    first = jnp.logical_and(pl.program_id(0) == 0, pl.program_id(1) == 0)

    @pl.when(first)
    def _():
        st_r[...] = jnp.zeros_like(st_r)

    st_r[0:1, :] = st_r[0:1, :] + jnp.sum(h1, axis=0, keepdims=True)
    st_r[1:2, :] = st_r[1:2, :] + jnp.sum(h1 * h1, axis=0, keepdims=True)


def _p2(h1_r, st_r, g_r, be_r, w2_r, b2_r, h2_r, st2_r, *, count):
    inv = 1.0 / count
    mean = st_r[0:1, :] * inv
    var = st_r[1:2, :] * inv - mean * mean
    scale = g_r[...] * lax.rsqrt(var + 1e-5)
    shift = be_r[...] - mean * scale
    a1 = jnp.maximum(h1_r[...].astype(jnp.float32) * scale + shift, 0.0)
    h2 = jnp.dot(a1, w2_r[...], preferred_element_type=jnp.float32) + b2_r[...]
    h2_r[...] = h2.astype(jnp.bfloat16)

    @pl.when(pl.program_id(0) == 0)
    def _():
        st2_r[...] = jnp.zeros_like(st2_r)

    st2_r[0:1, :] = st2_r[0:1, :] + jnp.sum(h2, axis=0, keepdims=True)
    st2_r[1:2, :] = st2_r[1:2, :] + jnp.sum(h2 * h2, axis=0, keepdims=True)


def _p3(h2_r, st_r, g_r, be_r, o_r, *, count):
    inv = 1.0 / count
    mean = st_r[0:1, :] * inv
    var = st_r[1:2, :] * inv - mean * mean
    scale = g_r[...] * lax.rsqrt(var + 1e-5)
    shift = be_r[...] - mean * scale
    o_r[...] = jnp.maximum(h2_r[...].astype(jnp.float32) * scale + shift, 0.0)


def kernel(xyz1, xyz2, features1, features2, W1, b1, g1, be1, W2, b2, g2, be2):
    B, N, _ = xyz1.shape
    S = xyz2.shape[1]
    C1 = features1.shape[2]
    C2 = features2.shape[2]
    H = W1.shape[0]
    f32 = jnp.float32

    xyz2t = jnp.swapaxes(xyz2, 1, 2)
    w1a = W1[:, :C2].T
    w1b = W1[:, C2:].T
    w2t = W2.T

    BN_BLK = 512
    NB = N // BN_BLK

    h1, st1 = pl.pallas_call(
        functools.partial(_p1, S=S),
        grid=(B, NB),
        in_specs=[
            pl.BlockSpec((1, BN_BLK, 3), lambda b, i: (b, i, 0)),
            pl.BlockSpec((1, 3, S), lambda b, i: (b, 0, 0)),
            pl.BlockSpec((1, BN_BLK, C1), lambda b, i: (b, i, 0)),
            pl.BlockSpec((1, S, C2), lambda b, i: (b, 0, 0)),
            pl.BlockSpec((C2, H), lambda b, i: (0, 0)),
            pl.BlockSpec((C1, H), lambda b, i: (0, 0)),
            pl.BlockSpec((1, H), lambda b, i: (0, 0)),
        ],
        out_specs=[
            pl.BlockSpec((1, BN_BLK, H), lambda b, i: (b, i, 0)),
            pl.BlockSpec((2, H), lambda b, i: (0, 0)),
        ],
        out_shape=[
            jax.ShapeDtypeStruct((B, N, H), jnp.bfloat16),
            jax.ShapeDtypeStruct((2, H), f32),
        ],
    )(xyz1, xyz2t, features1, features2, w1a, w1b, b1.reshape(1, H))

    R = B * N
    RB = 8192
    h1f = h1.reshape(R, H)
    count = float(R)

    h2, st2 = pl.pallas_call(
        functools.partial(_p2, count=count),
        grid=(R // RB,),
        in_specs=[
            pl.BlockSpec((RB, H), lambda i: (i, 0)),
            pl.BlockSpec((2, H), lambda i: (0, 0)),
            pl.BlockSpec((1, H), lambda i: (0, 0)),
            pl.BlockSpec((1, H), lambda i: (0, 0)),
            pl.BlockSpec((H, H), lambda i: (0, 0)),
            pl.BlockSpec((1, H), lambda i: (0, 0)),
        ],
        out_specs=[
            pl.BlockSpec((RB, H), lambda i: (i, 0)),
            pl.BlockSpec((2, H), lambda i: (0, 0)),
        ],
        out_shape=[
            jax.ShapeDtypeStruct((R, H), jnp.bfloat16),
            jax.ShapeDtypeStruct((2, H), f32),
        ],
    )(h1f, st1, g1.reshape(1, H), be1.reshape(1, H), w2t, b2.reshape(1, H))

    out = pl.pallas_call(
        functools.partial(_p3, count=count),
        grid=(R // RB,),
        in_specs=[
            pl.BlockSpec((RB, H), lambda i: (i, 0)),
            pl.BlockSpec((2, H), lambda i: (0, 0)),
            pl.BlockSpec((1, H), lambda i: (0, 0)),
            pl.BlockSpec((1, H), lambda i: (0, 0)),
        ],
        out_specs=pl.BlockSpec((RB, H), lambda i: (i, 0)),
        out_shape=jax.ShapeDtypeStruct((R, H), f32),
    )(h2, st2, g2.reshape(1, H), be2.reshape(1, H))

    return out.reshape(B, N, H)


# speedup vs baseline: 37.3051x; 1.0044x over previous
import functools

import jax
import jax.numpy as jnp
from jax import lax
from jax.experimental import pallas as pl


def _p1(xyz1_r, xyz2t_r, f1_r, f2_r, w1a_r, w1b_r, b1_r, h1_r, st_r, *, S):
    x1 = xyz1_r[0]
    x2t = xyz2t_r[0]
    xa, xb, xc = x1[:, 0:1], x1[:, 1:2], x1[:, 2:3]
    ya, yb, yc = x2t[0:1, :], x2t[1:2, :], x2t[2:3, :]
    sq1 = xa * xa + xb * xb + xc * xc
    sq2 = ya * ya + yb * yb + yc * yc
    dot2 = jnp.dot(x1 * -2.0, x2t, preferred_element_type=jnp.float32)
    dist = jnp.maximum(sq1 + sq2 + dot2, 0.0)

    bn = dist.shape[0]
    inf = jnp.float32(jnp.inf)

    LW = 128
    nch = S // LW
    a = dist[:, 0:LW]
    b = jnp.full_like(a, inf)
    c = jnp.full_like(a, inf)
    for j in range(1, nch):
        d = dist[:, j * LW:(j + 1) * LW]
        lo = jnp.minimum(a, d)
        hi = jnp.maximum(a, d)
        a = lo
        lo2 = jnp.minimum(b, hi)
        hi2 = jnp.maximum(b, hi)
        b = lo2
        c = jnp.minimum(c, hi2)

    i128 = lax.broadcasted_iota(jnp.int32, (bn, LW), 1).astype(jnp.float32)
    big = jnp.float32(float(S))
    m1 = jnp.min(a, axis=1, keepdims=True)
    p1 = jnp.min(jnp.where(a == m1, i128, big), axis=1, keepdims=True)
    rm1 = i128 == p1
    a1 = jnp.where(rm1, b, a)
    b1v = jnp.where(rm1, c, b)
    m2 = jnp.min(a1, axis=1, keepdims=True)
    p2 = jnp.min(jnp.where(a1 == m2, i128, big), axis=1, keepdims=True)
    rm2 = i128 == p2
    a2 = jnp.where(rm2, b1v, a1)
    m3 = jnp.min(a2, axis=1, keepdims=True)

    w1 = 1.0 / (m1 + 1e-8)
    w2 = 1.0 / (m2 + 1e-8)
    w3 = 1.0 / (m3 + 1e-8)
    wsum = w1 + w2 + w3
    nw1, nw2, nw3 = w1 / wsum, w2 / wsum, w3 / wsum

    a_mat = jnp.where(dist == m1, nw1, jnp.where(
        dist == m2, nw2, jnp.where(dist == m3, nw3, 0.0)))

    interp = jnp.dot(a_mat, f2_r[0], preferred_element_type=jnp.float32)
    h1 = (jnp.dot(interp, w1a_r[...], preferred_element_type=jnp.float32)
          + jnp.dot(f1_r[0], w1b_r[...], preferred_element_type=jnp.float32)
          + b1_r[...])
    h1_r[0] = h1.astype(jnp.bfloat16)

    first = jnp.logical_and(pl.program_id(0) == 0, pl.program_id(1) == 0)

    @pl.when(first)
    def _():
        st_r[...] = jnp.zeros_like(st_r)

    st_r[0:1, :] = st_r[0:1, :] + jnp.sum(h1, axis=0, keepdims=True)
    st_r[1:2, :] = st_r[1:2, :] + jnp.sum(h1 * h1, axis=0, keepdims=True)


def _p2(h1_r, st_r, g_r, be_r, w2_r, b2_r, h2_r, st2_r, *, count):
    inv = 1.0 / count
    mean = st_r[0:1, :] * inv
    var = st_r[1:2, :] * inv - mean * mean
    scale = g_r[...] * lax.rsqrt(var + 1e-5)
    shift = be_r[...] - mean * scale
    a1 = jnp.maximum(h1_r[...].astype(jnp.float32) * scale + shift, 0.0)
    h2 = jnp.dot(a1, w2_r[...], preferred_element_type=jnp.float32) + b2_r[...]
    h2_r[...] = h2.astype(jnp.bfloat16)

    @pl.when(pl.program_id(0) == 0)
    def _():
        st2_r[...] = jnp.zeros_like(st2_r)

    st2_r[0:1, :] = st2_r[0:1, :] + jnp.sum(h2, axis=0, keepdims=True)
    st2_r[1:2, :] = st2_r[1:2, :] + jnp.sum(h2 * h2, axis=0, keepdims=True)


def _p3(h2_r, st_r, g_r, be_r, o_r, *, count):
    inv = 1.0 / count
    mean = st_r[0:1, :] * inv
    var = st_r[1:2, :] * inv - mean * mean
    scale = g_r[...] * lax.rsqrt(var + 1e-5)
    shift = be_r[...] - mean * scale
    o_r[...] = jnp.maximum(h2_r[...].astype(jnp.float32) * scale + shift, 0.0)


def kernel(xyz1, xyz2, features1, features2, W1, b1, g1, be1, W2, b2, g2, be2):
    B, N, _ = xyz1.shape
    S = xyz2.shape[1]
    C1 = features1.shape[2]
    C2 = features2.shape[2]
    H = W1.shape[0]
    f32 = jnp.float32

    xyz2t = jnp.swapaxes(xyz2, 1, 2)
    w1a = W1[:, :C2].T
    w1b = W1[:, C2:].T
    w2t = W2.T

    BN_BLK = 512
    NB = N // BN_BLK

    h1, st1 = pl.pallas_call(
        functools.partial(_p1, S=S),
        grid=(B, NB),
        in_specs=[
            pl.BlockSpec((1, BN_BLK, 3), lambda b, i: (b, i, 0)),
            pl.BlockSpec((1, 3, S), lambda b, i: (b, 0, 0)),
            pl.BlockSpec((1, BN_BLK, C1), lambda b, i: (b, i, 0)),
            pl.BlockSpec((1, S, C2), lambda b, i: (b, 0, 0)),
            pl.BlockSpec((C2, H), lambda b, i: (0, 0)),
            pl.BlockSpec((C1, H), lambda b, i: (0, 0)),
            pl.BlockSpec((1, H), lambda b, i: (0, 0)),
        ],
        out_specs=[
            pl.BlockSpec((1, BN_BLK, H), lambda b, i: (b, i, 0)),
            pl.BlockSpec((2, H), lambda b, i: (0, 0)),
        ],
        out_shape=[
            jax.ShapeDtypeStruct((B, N, H), jnp.bfloat16),
            jax.ShapeDtypeStruct((2, H), f32),
        ],
    )(xyz1, xyz2t, features1, features2, w1a, w1b, b1.reshape(1, H))

    R = B * N
    RB = 16384
    h1f = h1.reshape(R, H)
    count = float(R)

    h2, st2 = pl.pallas_call(
        functools.partial(_p2, count=count),
        grid=(R // RB,),
        in_specs=[
            pl.BlockSpec((RB, H), lambda i: (i, 0)),
            pl.BlockSpec((2, H), lambda i: (0, 0)),
            pl.BlockSpec((1, H), lambda i: (0, 0)),
            pl.BlockSpec((1, H), lambda i: (0, 0)),
            pl.BlockSpec((H, H), lambda i: (0, 0)),
            pl.BlockSpec((1, H), lambda i: (0, 0)),
        ],
        out_specs=[
            pl.BlockSpec((RB, H), lambda i: (i, 0)),
            pl.BlockSpec((2, H), lambda i: (0, 0)),
        ],
        out_shape=[
            jax.ShapeDtypeStruct((R, H), jnp.bfloat16),
            jax.ShapeDtypeStruct((2, H), f32),
        ],
    )(h1f, st1, g1.reshape(1, H), be1.reshape(1, H), w2t, b2.reshape(1, H))

    out = pl.pallas_call(
        functools.partial(_p3, count=count),
        grid=(R // RB,),
        in_specs=[
            pl.BlockSpec((RB, H), lambda i: (i, 0)),
            pl.BlockSpec((2, H), lambda i: (0, 0)),
            pl.BlockSpec((1, H), lambda i: (0, 0)),
            pl.BlockSpec((1, H), lambda i: (0, 0)),
        ],
        out_specs=pl.BlockSpec((RB, H), lambda i: (i, 0)),
        out_shape=jax.ShapeDtypeStruct((R, H), f32),
    )(h2, st2, g2.reshape(1, H), be2.reshape(1, H))

    return out.reshape(B, N, H)
